# Initial kernel scaffold; baseline (speedup 1.0000x reference)
import jax
import jax.numpy as jnp
from jax.experimental import pallas as pl


def kernel(tokens, connections, tables):
    raise NotImplementedError("write your pallas kernel here")


# speedup vs baseline: 457.9209x; 457.9209x over previous
import functools

import jax
import jax.numpy as jnp
import numpy as np
from jax import lax
from jax.experimental import pallas as pl
from jax.experimental.pallas import tpu as pltpu
from jax.experimental.pallas import tpu_sc as plsc

SEQ = 16
NUM_HEADS = 16
IN_BITS = 256
NB = 12
NROWS = NUM_HEADS * IN_BITS
TBL = 2 ** NB
NW = 32
LPW = NROWS // NW
NPAIR = SEQ * (SEQ + 1) // 2
_TRI = [i * (i + 1) // 2 for i in range(SEQ)]

_PENCF = np.zeros((SEQ, 128), np.float32)
for _t in range(SEQ):
    for _b in range(4):
        _PENCF[_t, _b] = ((SEQ - 1 - _t) >> _b) & 1

_NCOL = 4
_CB = NROWS // _NCOL


def _prep_body(tok_ref, conn_ref, penc_ref, addr_ref):
    conn = conn_ref[...]
    iota = lax.broadcasted_iota(jnp.int32, (640, _CB), 0)
    tlo32 = jnp.zeros((640, _CB), jnp.float32)
    thi32 = jnp.zeros((640, _CB), jnp.float32)
    for k in range(NB):
        ck = jnp.broadcast_to(conn[k:k + 1, :], (640, _CB))
        m = jnp.where(iota == ck, jnp.float32(2.0 ** k), jnp.float32(0))
        if k < 8:
            tlo32 = tlo32 + m
        else:
            thi32 = thi32 + m
    tlo = tlo32.astype(jnp.bfloat16)
    thi = thi32.astype(jnp.bfloat16)
    tok = tok_ref[...]
    penc = penc_ref[...]
    f32 = jnp.float32

    def mm(x, lo, hi):
        return (jnp.dot(x, lo, preferred_element_type=f32)
                + jnp.dot(x, hi, preferred_element_type=f32))

    aq = mm(tok, tlo[0:256], thi[0:256])
    ak = mm(tok, tlo[256:512], thi[256:512])
    apf = mm(penc, tlo[512:640], thi[512:640])

    col0 = pl.program_id(0) * _CB
    rowbase = (lax.broadcasted_iota(jnp.int32, (SEQ, _CB), 1) + col0) * TBL
    for i in range(SEQ):
        if i == SEQ - 1:
            apblk = apf
        else:
            apblk = jnp.concatenate([apf[SEQ - 1 - i:], apf[:SEQ - 1 - i]], axis=0)
        blk = aq[i:i + 1] + ak + apblk
        addr_ref[i] = blk.astype(jnp.int32) + rowbase


_prep = pl.pallas_call(
    _prep_body,
    grid=(_NCOL,),
    in_specs=[
        pl.BlockSpec((SEQ, IN_BITS), lambda c: (0, 0)),
        pl.BlockSpec((NB, _CB), lambda c: (0, c)),
        pl.BlockSpec((SEQ, 128), lambda c: (0, 0)),
    ],
    out_specs=pl.BlockSpec((SEQ, SEQ, _CB), lambda c: (0, 0, c)),
    out_shape=jax.ShapeDtypeStruct((SEQ, SEQ, NROWS), jnp.int32),
    compiler_params=pltpu.CompilerParams(vmem_limit_bytes=100 * 1024 * 1024),
)


def _sc_gather_body(addr_hbm, tables_hbm, out_hbm, idx_v, buf_v, acc_v, sem):
    wid = lax.axis_index("s") * 2 + lax.axis_index("c")
    pltpu.sync_copy(addr_hbm.at[wid], idx_v)

    maxq = 32
    handles = []
    for i in range(SEQ):
        for j in range(i + 1):
            t = len(handles)
            if t >= maxq:
                handles[t - maxq].wait()
            handles.append(pltpu.async_copy(
                tables_hbm.at[idx_v.at[i * SEQ + j]],
                buf_v.at[_TRI[i] + j], sem))
    for h in handles[len(handles) - maxq:]:
        h.wait()

    ng = LPW // 16
    for i in range(SEQ):
        base = _TRI[i]
        for g in range(ng):
            acc_v[i, pl.ds(g * 16, 16)] = buf_v[base, pl.ds(g * 16, 16)]
        if i > 0:
            def body(j, _, i=i, base=base):
                for g in range(ng):
                    sl = pl.ds(g * 16, 16)
                    acc_v[i, sl] = acc_v[i, sl] + buf_v[base + j, sl]
                return 0
            lax.fori_loop(1, i + 1, body, 0)
    for i in range(SEQ):
        for g in range(ng):
            sl = pl.ds(g * 16, 16)
            v = acc_v[i, sl].astype(jnp.int32)
            acc_v[i, sl] = jnp.bitwise_and(v, 1).astype(jnp.float32)
    pltpu.sync_copy(acc_v, out_hbm.at[wid])


@functools.lru_cache(maxsize=1)
def _make_sc_gather():
    mesh = plsc.VectorSubcoreMesh(core_axis_name="c", subcore_axis_name="s")
    return pl.kernel(
        _sc_gather_body,
        mesh=mesh,
        out_type=jax.ShapeDtypeStruct((NW, SEQ, LPW), jnp.float32),
        scratch_types=[
            pltpu.VMEM((SEQ * SEQ, LPW), jnp.int32),
            pltpu.VMEM((NPAIR, LPW), jnp.float32),
            pltpu.VMEM((SEQ, LPW), jnp.float32),
            pltpu.SemaphoreType.DMA,
        ],
    )


def _finish_body(p_ref, out_ref):
    x = p_ref[...]
    x4 = x.reshape(SEQ, 2, SEQ, LPW)
    s = jnp.sum(x4, axis=0)
    counts = jnp.concatenate([s[0], s[1]], axis=-1)
    out_ref[...] = (counts > float(NUM_HEADS // 2)).astype(jnp.float32)


_finish = pl.pallas_call(
    _finish_body,
    out_shape=jax.ShapeDtypeStruct((SEQ, IN_BITS), jnp.float32),
)


def kernel(tokens, connections, tables):
    tok = tokens.astype(jnp.bfloat16)
    conn_t = connections.reshape(NROWS, NB).T
    pencf = jnp.asarray(_PENCF, jnp.bfloat16)
    addr = _prep(tok, conn_t, pencf)
    addr_sc = addr.reshape(SEQ * SEQ, NW, LPW).transpose(1, 0, 2)
    tables_flat = tables.reshape(-1)
    par = _make_sc_gather()(addr_sc, tables_flat)
    return _finish(par)
